# baseline (speedup 1.0000x reference)
import functools

import jax
import jax.numpy as jnp
from jax import lax
from jax.experimental import pallas as pl
from jax.experimental.pallas import tpu as pltpu
from jax.experimental.pallas import tpu_sc as plsc

R = 128
N = 32768
K = 64
LN = 16
NV = N // LN
WV = 4
NW = NV // WV
NCORES = 2
NSUB = 16
NWORKERS = NCORES * NSUB
ROWS_PER_W = R // NWORKERS
CAPC = 4096
INF_BITS = 0x7F800000


def _popcnt(mask):
    return plsc.all_reduce_population_count(mask)


def _n_iters(lo0, hi0):
    width = jnp.maximum(jnp.max(hi0) - jnp.max(lo0), 1)
    wbits = lax.bitcast_convert_type(width.astype(jnp.float32), jnp.int32)
    return lax.shift_right_logical(wbits, 23) - 126


def _bisect_bits(count_ge, lo0, hi0, k):
    def body(_, lh):
        lo, hi = lh
        mid = lo + lax.shift_right_logical(hi - lo, 1)
        ok = count_ge(plsc.bitcast(mid, jnp.float32)) >= k
        return (jnp.where(ok, mid, lo), jnp.where(ok, hi, mid))

    lo, _ = lax.fori_loop(0, _n_iters(lo0, hi0), body, (lo0, hi0))
    return lo


def _select_tj(loadv, loadi, nv, lo0, hi0):
    zero = jnp.zeros((LN,), jnp.int32)

    def count_ge(midf):
        def b(i, a):
            return a + _popcnt(loadv(i) >= midf)
        return lax.fori_loop(0, nv, b, zero)

    tb = _bisect_bits(count_ge, zero + lo0, hi0, K)
    tf = plsc.bitcast(tb, jnp.float32)

    def bcnt(i, a):
        v = loadv(i)
        return (a[0] + _popcnt(v > tf), a[1] + _popcnt(v >= tf))

    ngt, nge = lax.fori_loop(0, nv, bcnt, (zero, zero))
    needed = K - ngt

    def no_tie(_):
        return jnp.full((LN,), N - 1, jnp.int32)

    def with_tie(_):
        def jbody(_, lh):
            lo, hi = lh
            mid = lo + lax.shift_right_logical(hi - lo, 1)

            def cb(i, a):
                return a + _popcnt((loadv(i) == tf) & (loadi(i) <= mid))

            cnt = lax.fori_loop(0, nv, cb, zero)
            ok = cnt >= needed
            return (jnp.where(ok, lo, mid), jnp.where(ok, mid, hi))

        _, jhi = lax.fori_loop(
            0, 15, jbody, (jnp.full((LN,), -1, jnp.int32),
                           jnp.full((LN,), N - 1, jnp.int32)))
        return jhi

    jcut = lax.cond(jnp.any(nge - ngt != needed), with_tie, no_tie, None)
    return tf, jcut


def _select_tj_static(cvs, cis, lo0, hi0):
    zero = jnp.zeros((LN,), jnp.int32)

    def count_ge(midf):
        c = zero
        for v in cvs:
            c = c + _popcnt(v >= midf)
        return c

    tb = _bisect_bits(count_ge, zero + lo0, hi0, K)
    tf = plsc.bitcast(tb, jnp.float32)

    ngt = zero
    nge = zero
    for v in cvs:
        ngt = ngt + _popcnt(v > tf)
        nge = nge + _popcnt(v >= tf)
    needed = K - ngt

    def no_tie(_):
        return jnp.full((LN,), N - 1, jnp.int32)

    def with_tie(_):
        def jbody(_, lh):
            lo, hi = lh
            mid = lo + lax.shift_right_logical(hi - lo, 1)
            cnt = zero
            for v, ix in zip(cvs, cis):
                cnt = cnt + _popcnt((v == tf) & (ix <= mid))
            ok = cnt >= needed
            return (jnp.where(ok, lo, mid), jnp.where(ok, mid, hi))

        _, jhi = lax.fori_loop(
            0, 15, jbody, (jnp.full((LN,), -1, jnp.int32),
                           jnp.full((LN,), N - 1, jnp.int32)))
        return jhi

    jcut = lax.cond(jnp.any(nge - ngt != needed), with_tie, no_tie, None)
    return tf, jcut


def _build(interpret=False):
    mesh = plsc.VectorSubcoreMesh(
        core_axis_name="c", subcore_axis_name="s",
        num_cores=NCORES, num_subcores=NSUB)

    @functools.partial(
        pl.kernel,
        out_type=jax.ShapeDtypeStruct((R, N), jnp.float32),
        mesh=mesh,
        scratch_types=[
            pltpu.VMEM((N,), jnp.float32),
            pltpu.VMEM((N,), jnp.float32),
            pltpu.VMEM((N,), jnp.float32),
            pltpu.VMEM((NW * LN,), jnp.float32),
            pltpu.VMEM((16 * LN,), jnp.float32),
            pltpu.VMEM((NW + LN,), jnp.int32),
            pltpu.VMEM((CAPC + LN,), jnp.float32),
            pltpu.VMEM((CAPC + LN,), jnp.int32),
            pltpu.SemaphoreType.DMA,
            pltpu.SemaphoreType.DMA,
            pltpu.SemaphoreType.DMA,
        ],
        compiler_params=pltpu.CompilerParams(needs_layout_passes=False),
        interpret=interpret,
    )
    def topk_kernel(x_hbm, out_hbm, row_a, row_b, out_v, rmt, gmr, wl, cv, ci,
                    sem_a, sem_b, sem_o):
        wid = lax.axis_index("s") * NCORES + lax.axis_index("c")
        iota = lax.iota(jnp.int32, LN)
        iota_nw = iota * NW
        zf = jnp.zeros((LN,), jnp.float32)
        zi = jnp.zeros((LN,), jnp.int32)

        def zb(i, c):
            for u in range(8):
                out_v[pl.ds((i * 8 + u) * LN, LN)] = zf
            return c
        lax.fori_loop(0, NV // 8, zb, 0)

        row0 = wid * ROWS_PER_W
        bufs = [(row_a, sem_a), (row_b, sem_b)]
        pltpu.make_async_copy(x_hbm.at[row0], row_a, sem_a).start()

        prev = None
        for r in range(ROWS_PER_W):
            row = row0 + r
            row_v, sem = bufs[r % 2]
            pltpu.make_async_copy(x_hbm.at[row], row_v, sem).wait()
            if r + 1 < ROWS_PER_W:
                nrow_v, nsem = bufs[(r + 1) % 2]
                pltpu.make_async_copy(x_hbm.at[row + 1], nrow_v, nsem).start()

            def p1j(j, c):
                def p1b(b, gm):
                    g = gm
                    for wu in range(8):
                        w = (j * 4 + b) * 8 + wu
                        base = w * WV * LN
                        v0 = row_v[pl.ds(base, LN)]
                        v1 = row_v[pl.ds(base + LN, LN)]
                        v2 = row_v[pl.ds(base + 2 * LN, LN)]
                        v3 = row_v[pl.ds(base + 3 * LN, LN)]
                        rmv = jnp.maximum(jnp.maximum(v0, v1),
                                          jnp.maximum(v2, v3))
                        plsc.store_scatter(rmt, [iota_nw + w], rmv)
                        g = jnp.maximum(g, rmv)
                    return g
                gm = lax.fori_loop(
                    0, 4, p1b, jnp.full((LN,), -jnp.inf, jnp.float32))
                gmr[pl.ds(j * LN, LN)] = jnp.maximum(gm, 0.0)
                return c
            lax.fori_loop(0, 16, p1j, 0)
            gms = [gmr[pl.ds(t * LN, LN)] for t in range(16)]

            gmin = gms[0]
            gmax = gms[0]
            for g in gms[1:]:
                gmin = jnp.minimum(gmin, g)
                gmax = jnp.maximum(gmax, g)
            rowmax = jnp.max(gmax)
            hi_t = plsc.bitcast(jnp.full((LN,), rowmax, jnp.float32),
                                jnp.int32) + 1
            lo_l = plsc.bitcast(
                jnp.full((LN,), jnp.min(gmin), jnp.float32), jnp.int32)

            def count_ge_gm(midf):
                c = zi
                for g in gms:
                    c = c + _popcnt(g >= midf)
                return c
            lb = _bisect_bits(count_ge_gm, lo_l, hi_t, K)
            lf = plsc.bitcast(lb, jnp.float32)
            ls = jnp.max(lb)

            if prev is not None:
                ptf, pjcut, pncc, puse = prev
                pltpu.make_async_copy(out_v, out_hbm.at[row - 1], sem_o).wait()

                def puz_cand(_):
                    def b(i, c):
                        v = cv[pl.ds(i * LN, LN)]
                        ix = ci[pl.ds(i * LN, LN)]
                        keep = (v > ptf) | ((v == ptf) & (ix <= pjcut))
                        plsc.store_scatter(out_v, [ix], zf, mask=keep)
                        return c
                    lax.fori_loop(0, pncc, b, 0)
                    return 0

                def puz_row(_):
                    def b(i, c):
                        out_v[pl.ds(i * LN, LN)] = zf
                        return c
                    lax.fori_loop(0, NV, b, 0)
                    return 0

                lax.cond(puse, puz_cand, puz_row, None)

            def wlb(c, carry):
                off = carry
                wm = rmt[pl.ds(c * LN, LN)]
                for l in range(1, 16):
                    wm = jnp.maximum(wm, rmt[pl.ds(l * NW + c * LN, LN)])
                m = wm >= lf
                plsc.store_compressed(wl.at[pl.ds(off[0], LN)],
                                      iota + c * LN, mask=m)
                return off + _popcnt(m)
            nw_v = lax.fori_loop(0, NW // LN, wlb, zi)
            nw = jnp.where(ls > 0, jnp.max(nw_v), 0)

            nwb = (nw + LN - 1) // LN

            def fbatch(hb, off):
                wv = wl[pl.ds(hb * LN, LN)]
                lanes_valid = (iota + hb * LN) < nw
                wbase = wv * (WV * LN)
                offs = off
                for t in range(WV * LN):
                    idx = wbase + t
                    v = plsc.load_gather(row_v, [idx], mask=lanes_valid)
                    m = (v >= lf) & lanes_valid
                    cs = plsc.cumsum(m.astype(jnp.int32))
                    pos = offs + cs - 1
                    sm = m & (pos < CAPC)
                    plsc.store_scatter(cv, [pos], v, mask=sm)
                    plsc.store_scatter(ci, [pos], idx, mask=sm)
                    offs = offs + _popcnt(m)
                return offs
            offv = lax.fori_loop(0, nwb, fbatch, zi)
            cnum = jnp.max(offv)

            for kz in range(4):
                cv[pl.ds(jnp.minimum(cnum + kz * LN, CAPC), LN)] = zf

            use_cand = (ls > 0) & (cnum <= CAPC)
            fast = (ls > 0) & (cnum <= 8 * LN)
            ncc = (jnp.minimum(cnum, CAPC) + LN - 1) // LN

            def fast_branch(_):
                cvs = [cv[pl.ds(t * LN, LN)] for t in range(8)]
                cis = [ci[pl.ds(t * LN, LN)] for t in range(8)]
                return _select_tj_static(cvs, cis, ls, hi_t)

            def cand_branch(_):
                return _select_tj(
                    lambda i: cv[pl.ds(i * LN, LN)],
                    lambda i: ci[pl.ds(i * LN, LN)],
                    ncc, ls, hi_t)

            def row_branch(_):
                return _select_tj(
                    lambda i: row_v[pl.ds(i * LN, LN)],
                    lambda i: iota + i * LN,
                    NV, 0, hi_t)

            def slow_branch(_):
                return lax.cond(use_cand, cand_branch, row_branch, None)

            tf, jcut = lax.cond(fast, fast_branch, slow_branch, None)

            def keep_mask(v, ix):
                return (v > tf) | ((v == tf) & (ix <= jcut))

            def sc_cand(_):
                def b(i, c):
                    v = cv[pl.ds(i * LN, LN)]
                    ix = ci[pl.ds(i * LN, LN)]
                    plsc.store_scatter(out_v, [ix], v, mask=keep_mask(v, ix))
                    return c
                lax.fori_loop(0, ncc, b, 0)
                return 0

            def sc_row(_):
                def b(i, c):
                    v = row_v[pl.ds(i * LN, LN)]
                    ix = iota + i * LN
                    plsc.store_scatter(out_v, [ix], v, mask=keep_mask(v, ix))
                    return c
                lax.fori_loop(0, NV, b, 0)
                return 0

            lax.cond(use_cand, sc_cand, sc_row, None)

            pltpu.make_async_copy(out_v, out_hbm.at[row], sem_o).start()
            prev = (tf, jcut, ncc, use_cand)

        pltpu.make_async_copy(out_v, out_hbm.at[row0 + ROWS_PER_W - 1],
                              sem_o).wait()

    return topk_kernel


@functools.cache
def _get_kernel(interpret=False):
    return _build(interpret=interpret)


def kernel(x):
    return _get_kernel()(x)


# speedup vs baseline: 1.2416x; 1.2416x over previous
import functools

import jax
import jax.numpy as jnp
from jax import lax
from jax.experimental import pallas as pl
from jax.experimental.pallas import tpu as pltpu
from jax.experimental.pallas import tpu_sc as plsc

R = 128
N = 32768
K = 64
LN = 16
NV = N // LN
WV = 4
NW = NV // WV
NCORES = 2
NSUB = 16
NWORKERS = NCORES * NSUB
ROWS_PER_W = R // NWORKERS
CAPC = 4096
INF_BITS = 0x7F800000


def _popcnt(mask):
    return plsc.all_reduce_population_count(mask)


def _n_iters(lo0, hi0):
    width = jnp.maximum(jnp.max(hi0) - jnp.max(lo0), 1)
    wbits = lax.bitcast_convert_type(width.astype(jnp.float32), jnp.int32)
    return lax.shift_right_logical(wbits, 23) - 126


def _bisect_bits(count_ge, lo0, hi0, k):
    def body(_, lh):
        lo, hi = lh
        mid = lo + lax.shift_right_logical(hi - lo, 1)
        ok = count_ge(plsc.bitcast(mid, jnp.float32)) >= k
        return (jnp.where(ok, mid, lo), jnp.where(ok, hi, mid))

    lo, _ = lax.fori_loop(0, _n_iters(lo0, hi0), body, (lo0, hi0))
    return lo


def _select_tj(loadv, loadi, nv, lo0, hi0):
    zero = jnp.zeros((LN,), jnp.int32)

    def count_ge(midf):
        def b(i, a):
            return a + _popcnt(loadv(i) >= midf)
        return lax.fori_loop(0, nv, b, zero)

    tb = _bisect_bits(count_ge, zero + lo0, hi0, K)
    tf = plsc.bitcast(tb, jnp.float32)

    def bcnt(i, a):
        v = loadv(i)
        return (a[0] + _popcnt(v > tf), a[1] + _popcnt(v >= tf))

    ngt, nge = lax.fori_loop(0, nv, bcnt, (zero, zero))
    needed = K - ngt

    def no_tie(_):
        return jnp.full((LN,), N - 1, jnp.int32)

    def with_tie(_):
        def jbody(_, lh):
            lo, hi = lh
            mid = lo + lax.shift_right_logical(hi - lo, 1)

            def cb(i, a):
                return a + _popcnt((loadv(i) == tf) & (loadi(i) <= mid))

            cnt = lax.fori_loop(0, nv, cb, zero)
            ok = cnt >= needed
            return (jnp.where(ok, lo, mid), jnp.where(ok, mid, hi))

        _, jhi = lax.fori_loop(
            0, 15, jbody, (jnp.full((LN,), -1, jnp.int32),
                           jnp.full((LN,), N - 1, jnp.int32)))
        return jhi

    jcut = lax.cond(jnp.any(nge - ngt != needed), with_tie, no_tie, None)
    return tf, jcut


def _select_tj_static(cvs, cis, lo0, hi0):
    zero = jnp.zeros((LN,), jnp.int32)

    def count_ge(midf):
        c = zero
        for v in cvs:
            c = c + _popcnt(v >= midf)
        return c

    tb = _bisect_bits(count_ge, zero + lo0, hi0, K)
    tf = plsc.bitcast(tb, jnp.float32)

    ngt = zero
    nge = zero
    for v in cvs:
        ngt = ngt + _popcnt(v > tf)
        nge = nge + _popcnt(v >= tf)
    needed = K - ngt

    def no_tie(_):
        return jnp.full((LN,), N - 1, jnp.int32)

    def with_tie(_):
        def jbody(_, lh):
            lo, hi = lh
            mid = lo + lax.shift_right_logical(hi - lo, 1)
            cnt = zero
            for v, ix in zip(cvs, cis):
                cnt = cnt + _popcnt((v == tf) & (ix <= mid))
            ok = cnt >= needed
            return (jnp.where(ok, lo, mid), jnp.where(ok, mid, hi))

        _, jhi = lax.fori_loop(
            0, 15, jbody, (jnp.full((LN,), -1, jnp.int32),
                           jnp.full((LN,), N - 1, jnp.int32)))
        return jhi

    jcut = lax.cond(jnp.any(nge - ngt != needed), with_tie, no_tie, None)
    return tf, jcut


def _build(interpret=False):
    mesh = plsc.VectorSubcoreMesh(
        core_axis_name="c", subcore_axis_name="s",
        num_cores=NCORES, num_subcores=NSUB)

    @functools.partial(
        pl.kernel,
        out_type=jax.ShapeDtypeStruct((R, N), jnp.float32),
        mesh=mesh,
        scratch_types=[
            pltpu.VMEM((N,), jnp.float32),
            pltpu.VMEM((N,), jnp.float32),
            pltpu.VMEM((N,), jnp.float32),
            pltpu.VMEM((NW * LN,), jnp.float32),
            pltpu.VMEM((16 * LN,), jnp.float32),
            pltpu.VMEM((NW + LN,), jnp.int32),
            pltpu.VMEM((CAPC + LN,), jnp.float32),
            pltpu.VMEM((CAPC + LN,), jnp.int32),
            pltpu.SemaphoreType.DMA,
            pltpu.SemaphoreType.DMA,
            pltpu.SemaphoreType.DMA,
        ],
        compiler_params=pltpu.CompilerParams(needs_layout_passes=False),
        interpret=interpret,
    )
    def topk_kernel(x_hbm, out_hbm, row_a, row_b, out_v, rmt, gmr, wl, cv, ci,
                    sem_a, sem_b, sem_o):
        wid = lax.axis_index("s") * NCORES + lax.axis_index("c")
        iota = lax.iota(jnp.int32, LN)
        iota_nw = iota * NW
        zf = jnp.zeros((LN,), jnp.float32)
        zi = jnp.zeros((LN,), jnp.int32)

        def zb(i, c):
            for u in range(8):
                out_v[pl.ds((i * 8 + u) * LN, LN)] = zf
            return c
        lax.fori_loop(0, NV // 8, zb, 0)

        row0 = wid * ROWS_PER_W
        bufs = [(row_a, sem_a), (row_b, sem_b)]
        pltpu.make_async_copy(x_hbm.at[row0], row_a, sem_a).start()

        prev = None
        for r in range(ROWS_PER_W):
            row = row0 + r
            row_v, sem = bufs[r % 2]
            pltpu.make_async_copy(x_hbm.at[row], row_v, sem).wait()
            if r + 1 < ROWS_PER_W:
                nrow_v, nsem = bufs[(r + 1) % 2]
                pltpu.make_async_copy(x_hbm.at[row + 1], nrow_v, nsem).start()

            def p1j(j, c):
                def p1b(b, gm):
                    g = gm
                    for wu in range(8):
                        w = (j * 4 + b) * 8 + wu
                        base = w * WV * LN
                        v0 = row_v[pl.ds(base, LN)]
                        v1 = row_v[pl.ds(base + LN, LN)]
                        v2 = row_v[pl.ds(base + 2 * LN, LN)]
                        v3 = row_v[pl.ds(base + 3 * LN, LN)]
                        rmv = jnp.maximum(jnp.maximum(v0, v1),
                                          jnp.maximum(v2, v3))
                        plsc.store_scatter(rmt, [iota_nw + w], rmv)
                        g = jnp.maximum(g, rmv)
                    return g
                gm = lax.fori_loop(
                    0, 4, p1b, jnp.full((LN,), -jnp.inf, jnp.float32))
                gmr[pl.ds(j * LN, LN)] = jnp.maximum(gm, 0.0)
                return c
            lax.fori_loop(0, 16, p1j, 0)
            gms = [gmr[pl.ds(t * LN, LN)] for t in range(16)]

            gmin = gms[0]
            gmax = gms[0]
            for g in gms[1:]:
                gmin = jnp.minimum(gmin, g)
                gmax = jnp.maximum(gmax, g)
            rowmax = jnp.max(gmax)
            hi_t = plsc.bitcast(jnp.full((LN,), rowmax, jnp.float32),
                                jnp.int32) + 1
            lo_l = plsc.bitcast(
                jnp.full((LN,), jnp.min(gmin), jnp.float32), jnp.int32)

            def count_ge_gm(midf):
                c = zi
                for g in gms:
                    c = c + _popcnt(g >= midf)
                return c
            lb = _bisect_bits(count_ge_gm, lo_l, hi_t, K)
            lf = plsc.bitcast(lb, jnp.float32)
            ls = jnp.max(lb)

            if prev is not None:
                ptf, pjcut, pncc, puse = prev
                pltpu.make_async_copy(out_v, out_hbm.at[row - 1], sem_o).wait()

                def puz_cand(_):
                    def b(i, c):
                        v = cv[pl.ds(i * LN, LN)]
                        ix = ci[pl.ds(i * LN, LN)]
                        keep = (v > ptf) | ((v == ptf) & (ix <= pjcut))
                        plsc.store_scatter(out_v, [ix], zf, mask=keep)
                        return c
                    lax.fori_loop(0, pncc, b, 0)
                    return 0

                def puz_row(_):
                    def b(i, c):
                        out_v[pl.ds(i * LN, LN)] = zf
                        return c
                    lax.fori_loop(0, NV, b, 0)
                    return 0

                lax.cond(puse, puz_cand, puz_row, None)

            def wlb(c, carry):
                off = carry
                wm = rmt[pl.ds(c * LN, LN)]
                for l in range(1, 16):
                    wm = jnp.maximum(wm, rmt[pl.ds(l * NW + c * LN, LN)])
                m = wm >= lf
                cs = plsc.cumsum(m.astype(jnp.int32))
                pos = off + cs - 1
                plsc.store_scatter(wl, [pos], iota + c * LN, mask=m)
                return off + _popcnt(m)
            nw_v = lax.fori_loop(0, NW // LN, wlb, zi)
            nw = jnp.where(ls > 0, jnp.max(nw_v), 0)

            def fbody(h, off):
                w = wl[pl.ds(h, LN)][0]
                base = w * (WV * LN)
                offs = off
                for u in range(WV):
                    v = row_v[pl.ds(base + u * LN, LN)]
                    m = v >= lf
                    osc = jnp.minimum(offs, CAPC)[0]
                    plsc.store_compressed(cv.at[pl.ds(osc, LN)], v, mask=m)
                    plsc.store_compressed(
                        ci.at[pl.ds(osc, LN)], iota + base + u * LN, mask=m)
                    offs = offs + _popcnt(m)
                return offs
            offv = lax.fori_loop(0, nw, fbody, zi)
            cnum = jnp.max(offv)

            for kz in range(4):
                cv[pl.ds(jnp.minimum(cnum + kz * LN, CAPC), LN)] = zf

            use_cand = (ls > 0) & (cnum <= CAPC)
            fast = (ls > 0) & (cnum <= 8 * LN)
            ncc = (jnp.minimum(cnum, CAPC) + LN - 1) // LN

            def fast_branch(_):
                cvs = [cv[pl.ds(t * LN, LN)] for t in range(8)]
                cis = [ci[pl.ds(t * LN, LN)] for t in range(8)]
                return _select_tj_static(cvs, cis, ls, hi_t)

            def cand_branch(_):
                return _select_tj(
                    lambda i: cv[pl.ds(i * LN, LN)],
                    lambda i: ci[pl.ds(i * LN, LN)],
                    ncc, ls, hi_t)

            def row_branch(_):
                return _select_tj(
                    lambda i: row_v[pl.ds(i * LN, LN)],
                    lambda i: iota + i * LN,
                    NV, 0, hi_t)

            def slow_branch(_):
                return lax.cond(use_cand, cand_branch, row_branch, None)

            tf, jcut = lax.cond(fast, fast_branch, slow_branch, None)

            def keep_mask(v, ix):
                return (v > tf) | ((v == tf) & (ix <= jcut))

            def sc_cand(_):
                def b(i, c):
                    v = cv[pl.ds(i * LN, LN)]
                    ix = ci[pl.ds(i * LN, LN)]
                    plsc.store_scatter(out_v, [ix], v, mask=keep_mask(v, ix))
                    return c
                lax.fori_loop(0, ncc, b, 0)
                return 0

            def sc_row(_):
                def b(i, c):
                    v = row_v[pl.ds(i * LN, LN)]
                    ix = iota + i * LN
                    plsc.store_scatter(out_v, [ix], v, mask=keep_mask(v, ix))
                    return c
                lax.fori_loop(0, NV, b, 0)
                return 0

            lax.cond(use_cand, sc_cand, sc_row, None)

            pltpu.make_async_copy(out_v, out_hbm.at[row], sem_o).start()
            prev = (tf, jcut, ncc, use_cand)

        pltpu.make_async_copy(out_v, out_hbm.at[row0 + ROWS_PER_W - 1],
                              sem_o).wait()

    return topk_kernel


@functools.cache
def _get_kernel(interpret=False):
    return _build(interpret=interpret)


def kernel(x):
    return _get_kernel()(x)


# speedup vs baseline: 1.2522x; 1.0086x over previous
import functools

import jax
import jax.numpy as jnp
from jax import lax
from jax.experimental import pallas as pl
from jax.experimental.pallas import tpu as pltpu
from jax.experimental.pallas import tpu_sc as plsc

R = 128
N = 32768
K = 64
LN = 16
NV = N // LN
WV = 4
NW = NV // WV
NCORES = 2
NSUB = 16
NWORKERS = NCORES * NSUB
ROWS_PER_W = R // NWORKERS
CAPC = 4096
INF_BITS = 0x7F800000


def _popcnt(mask):
    return plsc.all_reduce_population_count(mask)


def _n_iters(lo0, hi0):
    width = jnp.maximum(jnp.max(hi0) - jnp.max(lo0), 1)
    wbits = lax.bitcast_convert_type(width.astype(jnp.float32), jnp.int32)
    return lax.shift_right_logical(wbits, 23) - 126


def _bisect_bits(count_ge, lo0, hi0, k):
    def body(_, lh):
        lo, hi = lh
        mid = lo + lax.shift_right_logical(hi - lo, 1)
        ok = count_ge(plsc.bitcast(mid, jnp.float32)) >= k
        return (jnp.where(ok, mid, lo), jnp.where(ok, hi, mid))

    lo, _ = lax.fori_loop(0, _n_iters(lo0, hi0), body, (lo0, hi0))
    return lo


def _select_tj(loadv, loadi, nv, lo0, hi0):
    zero = jnp.zeros((LN,), jnp.int32)

    def count_ge(midf):
        def b(i, a):
            return a + _popcnt(loadv(i) >= midf)
        return lax.fori_loop(0, nv, b, zero)

    tb = _bisect_bits(count_ge, zero + lo0, hi0, K)
    tf = plsc.bitcast(tb, jnp.float32)

    def bcnt(i, a):
        v = loadv(i)
        return (a[0] + _popcnt(v > tf), a[1] + _popcnt(v >= tf))

    ngt, nge = lax.fori_loop(0, nv, bcnt, (zero, zero))
    needed = K - ngt

    def no_tie(_):
        return jnp.full((LN,), N - 1, jnp.int32)

    def with_tie(_):
        def jbody(_, lh):
            lo, hi = lh
            mid = lo + lax.shift_right_logical(hi - lo, 1)

            def cb(i, a):
                return a + _popcnt((loadv(i) == tf) & (loadi(i) <= mid))

            cnt = lax.fori_loop(0, nv, cb, zero)
            ok = cnt >= needed
            return (jnp.where(ok, lo, mid), jnp.where(ok, mid, hi))

        _, jhi = lax.fori_loop(
            0, 15, jbody, (jnp.full((LN,), -1, jnp.int32),
                           jnp.full((LN,), N - 1, jnp.int32)))
        return jhi

    jcut = lax.cond(jnp.any(nge - ngt != needed), with_tie, no_tie, None)
    return tf, jcut


def _select_tj_static(cvs, cis, lo0, hi0):
    zero = jnp.zeros((LN,), jnp.int32)

    def count_ge(midf):
        c = zero
        for v in cvs:
            c = c + _popcnt(v >= midf)
        return c

    tb = _bisect_bits(count_ge, zero + lo0, hi0, K)
    tf = plsc.bitcast(tb, jnp.float32)

    ngt = zero
    nge = zero
    for v in cvs:
        ngt = ngt + _popcnt(v > tf)
        nge = nge + _popcnt(v >= tf)
    needed = K - ngt

    def no_tie(_):
        return jnp.full((LN,), N - 1, jnp.int32)

    def with_tie(_):
        def jbody(_, lh):
            lo, hi = lh
            mid = lo + lax.shift_right_logical(hi - lo, 1)
            cnt = zero
            for v, ix in zip(cvs, cis):
                cnt = cnt + _popcnt((v == tf) & (ix <= mid))
            ok = cnt >= needed
            return (jnp.where(ok, lo, mid), jnp.where(ok, mid, hi))

        _, jhi = lax.fori_loop(
            0, 15, jbody, (jnp.full((LN,), -1, jnp.int32),
                           jnp.full((LN,), N - 1, jnp.int32)))
        return jhi

    jcut = lax.cond(jnp.any(nge - ngt != needed), with_tie, no_tie, None)
    return tf, jcut


def _build(interpret=False):
    mesh = plsc.VectorSubcoreMesh(
        core_axis_name="c", subcore_axis_name="s",
        num_cores=NCORES, num_subcores=NSUB)

    @functools.partial(
        pl.kernel,
        out_type=jax.ShapeDtypeStruct((R, N), jnp.float32),
        mesh=mesh,
        scratch_types=[
            pltpu.VMEM((N,), jnp.float32),
            pltpu.VMEM((N,), jnp.float32),
            pltpu.VMEM((N,), jnp.float32),
            pltpu.VMEM((NW * LN,), jnp.float32),
            pltpu.VMEM((16 * LN,), jnp.float32),
            pltpu.VMEM((NW + LN,), jnp.int32),
            pltpu.VMEM((CAPC + LN,), jnp.float32),
            pltpu.VMEM((CAPC + LN,), jnp.int32),
            pltpu.SemaphoreType.DMA,
            pltpu.SemaphoreType.DMA,
            pltpu.SemaphoreType.DMA,
        ],
        compiler_params=pltpu.CompilerParams(needs_layout_passes=False),
        interpret=interpret,
    )
    def topk_kernel(x_hbm, out_hbm, row_a, row_b, out_v, rmt, gmr, wl, cv, ci,
                    sem_a, sem_b, sem_o):
        wid = lax.axis_index("s") * NCORES + lax.axis_index("c")
        iota = lax.iota(jnp.int32, LN)
        iota_nw = iota * NW
        zf = jnp.zeros((LN,), jnp.float32)
        zi = jnp.zeros((LN,), jnp.int32)

        def zb(i, c):
            for u in range(8):
                out_v[pl.ds((i * 8 + u) * LN, LN)] = zf
            return c
        lax.fori_loop(0, NV // 8, zb, 0)

        row0 = wid * ROWS_PER_W
        bufs = [(row_a, sem_a), (row_b, sem_b)]
        pltpu.make_async_copy(x_hbm.at[row0], row_a, sem_a).start()

        prev = None
        for r in range(ROWS_PER_W):
            row = row0 + r
            row_v, sem = bufs[r % 2]
            pltpu.make_async_copy(x_hbm.at[row], row_v, sem).wait()
            if r + 1 < ROWS_PER_W:
                nrow_v, nsem = bufs[(r + 1) % 2]
                pltpu.make_async_copy(x_hbm.at[row + 1], nrow_v, nsem).start()

            def p1j(j, c):
                def p1b(b, gm):
                    g = gm
                    for wu in range(8):
                        w = (j * 4 + b) * 8 + wu
                        base = w * WV * LN
                        v0 = row_v[pl.ds(base, LN)]
                        v1 = row_v[pl.ds(base + LN, LN)]
                        v2 = row_v[pl.ds(base + 2 * LN, LN)]
                        v3 = row_v[pl.ds(base + 3 * LN, LN)]
                        rmv = jnp.maximum(jnp.maximum(v0, v1),
                                          jnp.maximum(v2, v3))
                        plsc.store_scatter(rmt, [iota_nw + w], rmv)
                        g = jnp.maximum(g, rmv)
                    return g
                gm = lax.fori_loop(
                    0, 4, p1b, jnp.full((LN,), -jnp.inf, jnp.float32))
                gmr[pl.ds(j * LN, LN)] = jnp.maximum(gm, 0.0)
                return c
            lax.fori_loop(0, 16, p1j, 0)
            gms = [gmr[pl.ds(t * LN, LN)] for t in range(16)]

            gmin = gms[0]
            gmax = gms[0]
            for g in gms[1:]:
                gmin = jnp.minimum(gmin, g)
                gmax = jnp.maximum(gmax, g)
            rowmax = jnp.max(gmax)
            hi_t = plsc.bitcast(jnp.full((LN,), rowmax, jnp.float32),
                                jnp.int32) + 1
            lo_l = plsc.bitcast(
                jnp.full((LN,), jnp.min(gmin), jnp.float32), jnp.int32)

            def count_ge_gm(midf):
                c = zi
                for g in gms:
                    c = c + _popcnt(g >= midf)
                return c
            lb = _bisect_bits(count_ge_gm, lo_l, hi_t, K)
            lf = plsc.bitcast(lb, jnp.float32)
            ls = jnp.max(lb)

            if prev is not None:
                ptf, pjcut, pncc, puse = prev
                pltpu.make_async_copy(out_v, out_hbm.at[row - 1], sem_o).wait()

                def puz_cand(_):
                    def b(i, c):
                        v = cv[pl.ds(i * LN, LN)]
                        ix = ci[pl.ds(i * LN, LN)]
                        keep = (v > ptf) | ((v == ptf) & (ix <= pjcut))
                        plsc.store_scatter(out_v, [ix], zf, mask=keep)
                        return c
                    lax.fori_loop(0, pncc, b, 0)
                    return 0

                def puz_row(_):
                    def b(i, c):
                        out_v[pl.ds(i * LN, LN)] = zf
                        return c
                    lax.fori_loop(0, NV, b, 0)
                    return 0

                lax.cond(puse, puz_cand, puz_row, None)

            def wlb(c, carry):
                off = carry
                wm = rmt[pl.ds(c * LN, LN)]
                for l in range(1, 16):
                    wm = jnp.maximum(wm, rmt[pl.ds(l * NW + c * LN, LN)])
                m = wm >= lf
                plsc.store_compressed(wl.at[pl.ds(off[0], LN)],
                                      iota + c * LN, mask=m)
                return off + _popcnt(m)
            nw_v = lax.fori_loop(0, NW // LN, wlb, zi)
            nw = jnp.where(ls > 0, jnp.max(nw_v), 0)

            def fbody(h, off):
                w = wl[pl.ds(h, LN)][0]
                base = w * (WV * LN)
                offs = off
                for u in range(WV):
                    v = row_v[pl.ds(base + u * LN, LN)]
                    m = v >= lf
                    osc = jnp.minimum(offs, CAPC)[0]
                    plsc.store_compressed(cv.at[pl.ds(osc, LN)], v, mask=m)
                    plsc.store_compressed(
                        ci.at[pl.ds(osc, LN)], iota + base + u * LN, mask=m)
                    offs = offs + _popcnt(m)
                return offs
            offv = lax.fori_loop(0, nw, fbody, zi)
            cnum = jnp.max(offv)

            for kz in range(4):
                cv[pl.ds(jnp.minimum(cnum + kz * LN, CAPC), LN)] = zf

            use_cand = (ls > 0) & (cnum <= CAPC)
            fast = (ls > 0) & (cnum <= 8 * LN)
            ncc = (jnp.minimum(cnum, CAPC) + LN - 1) // LN

            def fast_branch(_):
                cvs = [cv[pl.ds(t * LN, LN)] for t in range(8)]
                cis = [ci[pl.ds(t * LN, LN)] for t in range(8)]
                return _select_tj_static(cvs, cis, ls, hi_t)

            def cand_branch(_):
                return _select_tj(
                    lambda i: cv[pl.ds(i * LN, LN)],
                    lambda i: ci[pl.ds(i * LN, LN)],
                    ncc, ls, hi_t)

            def row_branch(_):
                return _select_tj(
                    lambda i: row_v[pl.ds(i * LN, LN)],
                    lambda i: iota + i * LN,
                    NV, 0, hi_t)

            def slow_branch(_):
                return lax.cond(use_cand, cand_branch, row_branch, None)

            tf, jcut = lax.cond(fast, fast_branch, slow_branch, None)

            def keep_mask(v, ix):
                return (v > tf) | ((v == tf) & (ix <= jcut))

            def sc_cand(_):
                def b(i, c):
                    v = cv[pl.ds(i * LN, LN)]
                    ix = ci[pl.ds(i * LN, LN)]
                    plsc.store_scatter(out_v, [ix], v, mask=keep_mask(v, ix))
                    return c
                lax.fori_loop(0, ncc, b, 0)
                return 0

            def sc_row(_):
                def b(i, c):
                    v = row_v[pl.ds(i * LN, LN)]
                    ix = iota + i * LN
                    plsc.store_scatter(out_v, [ix], v, mask=keep_mask(v, ix))
                    return c
                lax.fori_loop(0, NV, b, 0)
                return 0

            lax.cond(use_cand, sc_cand, sc_row, None)

            pltpu.make_async_copy(out_v, out_hbm.at[row], sem_o).start()
            prev = (tf, jcut, ncc, use_cand)

        pltpu.make_async_copy(out_v, out_hbm.at[row0 + ROWS_PER_W - 1],
                              sem_o).wait()

    return topk_kernel


@functools.cache
def _get_kernel(interpret=False):
    return _build(interpret=interpret)


def kernel(x):
    return _get_kernel()(x)
